# baseline (speedup 1.0000x reference)
import functools

import jax
import jax.numpy as jnp
from jax import lax
from jax.experimental import pallas as pl
from jax.experimental.pallas import tpu as pltpu
from jax.experimental.pallas import tpu_sc as plsc

HIDDEN = 128
NFILT = 128
NLAYERS = 6
NGAUSS = 50
CUTOFF = 10.0
NGRAPHS = 64

_LN2 = 0.6931471805599453
_STEP = CUTOFF / (NGAUSS - 1)
_COEFF = -0.5 / _STEP**2

_NC = 2
_NS = 16
_NW = _NC * _NS
_CH = 128

_MESH = plsc.VectorSubcoreMesh(core_axis_name="c", subcore_axis_name="s")


def _ssp(x):
    return jnp.maximum(x, 0.0) + jnp.log1p(jnp.exp(-jnp.abs(x))) - _LN2


def _make_posdiff(n_pad, e_pad):
    jw = e_pad // (_NW * _CH)
    assert jw % 3 == 0

    @functools.partial(
        pl.kernel,
        out_type=jax.ShapeDtypeStruct((e_pad, 16), jnp.float32),
        mesh=_MESH,
        scratch_types=[
            pltpu.VMEM((jw, _CH), jnp.int32),
            pltpu.VMEM((jw, _CH), jnp.int32),
            pltpu.VMEM((_CH, 16), jnp.float32),
            pltpu.VMEM((_CH, 16), jnp.float32),
            pltpu.SemaphoreType.DMA,
            pltpu.SemaphoreType.DMA,
        ],
        compiler_params=pltpu.CompilerParams(use_tc_tiling_on_sc=False),
    )
    def posdiff(pos_hbm, row_hbm, col_hbm, d3_hbm, ridx, cidx, pr0, pc0,
                sem1, sem2):
        c = lax.axis_index("c")
        s = lax.axis_index("s")
        w = s * _NC + c
        pltpu.sync_copy(row_hbm.at[w], ridx)
        pltpu.sync_copy(col_hbm.at[w], cidx)

        def chunk(j, carry):
            cp1 = pltpu.async_copy(pos_hbm.at[ridx.at[j]], pr0, sem1)
            cp2 = pltpu.async_copy(pos_hbm.at[cidx.at[j]], pc0, sem2)
            cp1.wait()
            cp2.wait()

            def row_body(i, carry2):
                pr0[i] = pr0[i] - pc0[i]
                return carry2

            lax.fori_loop(0, _CH, row_body, 0)
            base = (w * jw + j) * _CH
            pltpu.sync_copy(pr0, d3_hbm.at[pl.ds(base, _CH)])
            return carry

        lax.fori_loop(0, jw, chunk, 0)

    return posdiff


def _make_msg_scatter(n_pad, e_pad):
    half = n_pad // 2
    jt = e_pad // (_NS * _CH)
    rpt = half // _NS
    arows = half + 128

    assert jt % 2 == 0
    jh = jt // 2

    @functools.partial(
        pl.kernel,
        out_type=jax.ShapeDtypeStruct((_NC, half, HIDDEN), jnp.float32),
        mesh=_MESH,
        scratch_types=[
            pltpu.VMEM((jh, _CH), jnp.int32),
            pltpu.VMEM((jh, _CH), jnp.int32),
            pltpu.VMEM((1, _CH), jnp.int32),
            pltpu.VMEM((_CH, HIDDEN), jnp.float32),
            pltpu.VMEM((_CH, HIDDEN), jnp.float32),
            pltpu.VMEM_SHARED((arows, HIDDEN), jnp.float32),
            pltpu.SemaphoreType.DMA,
        ],
    )
    def msg_scatter(xl_hbm, w_hbm, row_hbm, col_hbm, zeros_hbm, agg_hbm,
                    ridx, cidx, cl0, xg0, wv, aggs, gsem):
        c = lax.axis_index("c")
        s = lax.axis_index("s")
        pltpu.sync_copy(zeros_hbm, aggs.at[pl.ds(s * rpt, rpt)])
        plsc.subcore_barrier()
        lo = c * half
        trash = half + s * 8

        def sweep(hs):
            base_j = hs * jh

            def wbase(jl):
                return (s * jt + base_j) * _CH + jl * _CH

            pltpu.sync_copy(row_hbm.at[s, hs], ridx)
            pltpu.sync_copy(col_hbm.at[s, hs], cidx)

            def do_chunk(jl, carry):
                cp = pltpu.async_copy(xl_hbm.at[ridx.at[jl]], xg0, gsem)
                pltpu.sync_copy(w_hbm.at[pl.ds(wbase(jl), _CH)], wv)

                def remap(k, carry2):
                    v = cidx[jl, pl.ds(k * 16, 16)] - lo
                    ok = (v >= 0) & (v < half)
                    cl0[0, pl.ds(k * 16, 16)] = jnp.where(ok, v, trash)
                    return carry2

                lax.fori_loop(0, _CH // 16, remap, 0, unroll=True)
                cp.wait()

                def row_body(i, carry2):
                    for k in range(HIDDEN // 16):
                        sl = pl.ds(k * 16, 16)
                        xg0[i, sl] = xg0[i, sl] * wv[i, sl]
                    return carry2

                lax.fori_loop(0, _CH, row_body, 0)
                pltpu.sync_copy(xg0, aggs.at[cl0.at[0]], add=True)
                return carry

            lax.fori_loop(0, jh, do_chunk, 0)

        sweep(0)
        sweep(1)
        plsc.subcore_barrier()
        pltpu.sync_copy(aggs.at[pl.ds(s * rpt, rpt)],
                        agg_hbm.at[c, pl.ds(s * rpt, rpt)])

    return msg_scatter


def _init_body(atom_ref, emb_ref, cf1_ref, h_ref, xl_ref):
    a = atom_ref[...].astype(jnp.float32)
    e0 = emb_ref[0:1, :]
    e1 = emb_ref[1:2, :]
    h = e0 + a * (e1 - e0)
    h_ref[...] = h
    xl_ref[...] = jnp.dot(h, cf1_ref[...], preferred_element_type=jnp.float32)


def _make_filter_body(e_real, be):
    def _filter_body(d3_ref, w1_ref, b1_ref, w2_ref, b2_ref, w_ref):
        i = pl.program_id(0)
        d = d3_ref[...]
        s = jnp.sum(d * d, axis=1, keepdims=True) + 1e-12
        dist = jnp.sqrt(s)
        cc = 0.5 * (jnp.cos(dist * (jnp.pi / CUTOFF)) + 1.0)
        erow = i * be + lax.broadcasted_iota(jnp.int32, (be, 1), 0)
        cc = jnp.where(erow < e_real, cc, 0.0)
        off = lax.broadcasted_iota(jnp.int32, (be, NGAUSS), 1).astype(
            jnp.float32) * _STEP
        attr = jnp.exp(_COEFF * (dist - off) ** 2)
        t = _ssp(jnp.dot(attr, w1_ref[...], preferred_element_type=jnp.float32)
                 + b1_ref[...])
        wmat = (jnp.dot(t, w2_ref[...], preferred_element_type=jnp.float32)
                + b2_ref[...])
        w_ref[...] = wmat * cc

    return _filter_body


def _node_body(h_ref, agg_ref, cf2w_ref, cf2b_ref, linw_ref, linb_ref,
               cf1n_ref, hout_ref, xlout_ref):
    a = agg_ref[0]
    t = _ssp(jnp.dot(a, cf2w_ref[...], preferred_element_type=jnp.float32)
             + cf2b_ref[...])
    t = jnp.dot(t, linw_ref[...], preferred_element_type=jnp.float32) + linb_ref[...]
    h = h_ref[...] + t
    hout_ref[...] = h
    xlout_ref[...] = jnp.dot(h, cf1n_ref[...], preferred_element_type=jnp.float32)


def _readout_body(h_ref, batch_ref, l1w_ref, l1b_ref, ew_ref, qw_ref,
                  eb_ref, qb_ref, e_ref, q_ref, et_ref, qt_ref):
    i = pl.program_id(0)
    hm = _ssp(jnp.dot(h_ref[...], l1w_ref[...], preferred_element_type=jnp.float32)
              + l1b_ref[...])
    e = jnp.dot(hm, ew_ref[...], preferred_element_type=jnp.float32) + eb_ref[...]
    q = jnp.dot(hm, qw_ref[...], preferred_element_type=jnp.float32) + qb_ref[...]
    e_ref[...] = e
    q_ref[...] = q
    b = batch_ref[...]
    gid = lax.broadcasted_iota(jnp.int32, (b.shape[0], NGRAPHS), 1)
    onehot = (b == gid).astype(jnp.float32)
    ec = jnp.sum(onehot * e, axis=0, keepdims=True)
    qc = jnp.sum(onehot * q, axis=0, keepdims=True)

    @pl.when(i == 0)
    def _():
        et_ref[...] = jnp.zeros_like(et_ref)
        qt_ref[...] = jnp.zeros_like(qt_ref)

    et_ref[...] += ec
    qt_ref[...] += qc


def _full(shape):
    return pl.BlockSpec(shape, lambda i: tuple(0 for _ in shape))


def kernel(node_atom, pos, batch, edge_index, emb, mlp_w1, mlp_b1, mlp_w2,
           mlp_b2, cf_lin1_w, cf_lin2_w, cf_lin2_b, lin_w, lin_b, lin1_w,
           lin1_b, e_w, e_b, q_w, q_b):
    n = pos.shape[0]
    e = edge_index.shape[1]
    n_pad = ((n + 2047) // 2048) * 2048
    e_pad = ((e + 12287) // 12288) * 12288
    half = n_pad // 2

    row = edge_index[0].astype(jnp.int32)
    col = edge_index[1].astype(jnp.int32)
    rowp = jnp.concatenate([row, jnp.zeros((e_pad - e,), jnp.int32)])
    colp = jnp.concatenate([col, jnp.zeros((e_pad - e,), jnp.int32)])
    row32 = rowp.reshape(_NW, e_pad // (_NW * _CH), _CH)
    col32 = colp.reshape(_NW, e_pad // (_NW * _CH), _CH)
    row16 = rowp.reshape(_NS, 2, e_pad // (2 * _NS * _CH), _CH)
    col16 = colp.reshape(_NS, 2, e_pad // (2 * _NS * _CH), _CH)
    pos_pad = jnp.zeros((n_pad, 16), jnp.float32).at[:n, :3].set(pos)
    atom_p = jnp.zeros((n_pad, 1), jnp.int32).at[:n, 0].set(
        node_atom.astype(jnp.int32))
    batch_p = jnp.full((n_pad, 1), NGRAPHS, jnp.int32).at[:n, 0].set(
        batch.astype(jnp.int32))
    zeros_sc = jnp.zeros((half // _NS, HIDDEN), jnp.float32)

    d3 = _make_posdiff(n_pad, e_pad)(pos_pad, row32, col32)
    msg_scatter = _make_msg_scatter(n_pad, e_pad)

    bn = 1024
    be = 2048
    gn = n_pad // bn
    ge = e_pad // be
    hb = half // bn

    h, xl = pl.pallas_call(
        _init_body,
        grid=(gn,),
        in_specs=[pl.BlockSpec((bn, 1), lambda i: (i, 0)),
                  _full((2, HIDDEN)), _full((HIDDEN, HIDDEN))],
        out_specs=[pl.BlockSpec((bn, HIDDEN), lambda i: (i, 0))] * 2,
        out_shape=[jax.ShapeDtypeStruct((n_pad, HIDDEN), jnp.float32)] * 2,
    )(atom_p, emb, cf_lin1_w[0])

    filter_call = pl.pallas_call(
        _make_filter_body(e, be),
        grid=(ge,),
        in_specs=[pl.BlockSpec((be, 16), lambda i: (i, 0)),
                  _full((NGAUSS, NFILT)), _full((1, NFILT)),
                  _full((NFILT, NFILT)), _full((1, NFILT))],
        out_specs=pl.BlockSpec((be, NFILT), lambda i: (i, 0)),
        out_shape=jax.ShapeDtypeStruct((e_pad, NFILT), jnp.float32),
    )

    node_call = pl.pallas_call(
        _node_body,
        grid=(gn,),
        in_specs=[pl.BlockSpec((bn, HIDDEN), lambda i: (i, 0)),
                  pl.BlockSpec((1, bn, HIDDEN), lambda i: (i // hb, i % hb, 0)),
                  _full((HIDDEN, HIDDEN)), _full((1, HIDDEN)),
                  _full((HIDDEN, HIDDEN)), _full((1, HIDDEN)),
                  _full((HIDDEN, HIDDEN))],
        out_specs=[pl.BlockSpec((bn, HIDDEN), lambda i: (i, 0))] * 2,
        out_shape=[jax.ShapeDtypeStruct((n_pad, HIDDEN), jnp.float32)] * 2,
    )

    for l in range(NLAYERS):
        wmat = filter_call(d3, mlp_w1[l], mlp_b1[l].reshape(1, NFILT),
                           mlp_w2[l], mlp_b2[l].reshape(1, NFILT))
        agg = msg_scatter(xl, wmat, row16, col16, zeros_sc)
        cf1n = cf_lin1_w[(l + 1) % NLAYERS]
        h, xl = node_call(h, agg, cf_lin2_w[l],
                          cf_lin2_b[l].reshape(1, HIDDEN), lin_w[l],
                          lin_b[l].reshape(1, HIDDEN), cf1n)

    hh = HIDDEN // 2
    e2, q2, et, qt = pl.pallas_call(
        _readout_body,
        grid=(gn,),
        in_specs=[pl.BlockSpec((bn, HIDDEN), lambda i: (i, 0)),
                  pl.BlockSpec((bn, 1), lambda i: (i, 0)),
                  _full((HIDDEN, hh)), _full((1, hh)),
                  _full((hh, 1)), _full((hh, 1)),
                  _full((1, 1)), _full((1, 1))],
        out_specs=[pl.BlockSpec((bn, 1), lambda i: (i, 0)),
                   pl.BlockSpec((bn, 1), lambda i: (i, 0)),
                   _full((1, NGRAPHS)), _full((1, NGRAPHS))],
        out_shape=[jax.ShapeDtypeStruct((n_pad, 1), jnp.float32),
                   jax.ShapeDtypeStruct((n_pad, 1), jnp.float32),
                   jax.ShapeDtypeStruct((1, NGRAPHS), jnp.float32),
                   jax.ShapeDtypeStruct((1, NGRAPHS), jnp.float32)],
    )(h, batch_p, lin1_w, lin1_b.reshape(1, hh), e_w, q_w,
      e_b.reshape(1, 1), q_b.reshape(1, 1))

    return (e2[:n, 0], q2[:n, 0], et[0], qt[0])


# speedup vs baseline: 1.5081x; 1.5081x over previous
import functools

import jax
import jax.numpy as jnp
from jax import lax
from jax.experimental import pallas as pl
from jax.experimental.pallas import tpu as pltpu
from jax.experimental.pallas import tpu_sc as plsc

HIDDEN = 128
NFILT = 128
NLAYERS = 6
NGAUSS = 50
CUTOFF = 10.0
NGRAPHS = 64

_LN2 = 0.6931471805599453
_STEP = CUTOFF / (NGAUSS - 1)
_COEFF = -0.5 / _STEP**2

_NC = 2
_NS = 16
_NW = _NC * _NS
_CH = 128

_MESH = plsc.VectorSubcoreMesh(core_axis_name="c", subcore_axis_name="s")


def _ssp(x):
    return jnp.maximum(x, 0.0) + jnp.log1p(jnp.exp(-jnp.abs(x))) - _LN2


def _make_posdiff(n_pad, e_pad):
    jw = e_pad // (_NW * _CH)

    @functools.partial(
        pl.kernel,
        out_type=jax.ShapeDtypeStruct((e_pad, 16), jnp.float32),
        mesh=_MESH,
        scratch_types=[
            pltpu.VMEM((jw, _CH), jnp.int32),
            pltpu.VMEM((jw, _CH), jnp.int32),
            pltpu.VMEM((_CH, 16), jnp.float32),
            pltpu.VMEM((_CH, 16), jnp.float32),
            pltpu.SemaphoreType.DMA,
            pltpu.SemaphoreType.DMA,
        ],
        compiler_params=pltpu.CompilerParams(use_tc_tiling_on_sc=False),
    )
    def posdiff(pos_hbm, row_hbm, col_hbm, d3_hbm, ridx, cidx, pr, pc, sem1, sem2):
        c = lax.axis_index("c")
        s = lax.axis_index("s")
        w = s * _NC + c
        pltpu.sync_copy(row_hbm.at[w], ridx)
        pltpu.sync_copy(col_hbm.at[w], cidx)

        def chunk(j, carry):
            cp1 = pltpu.async_copy(pos_hbm.at[ridx.at[j]], pr, sem1)
            cp2 = pltpu.async_copy(pos_hbm.at[cidx.at[j]], pc, sem2)
            cp1.wait()
            cp2.wait()

            def row_body(i, carry2):
                pr[i] = pr[i] - pc[i]
                return carry2

            lax.fori_loop(0, _CH, row_body, 0)
            base = (w * jw + j) * _CH
            pltpu.sync_copy(pr, d3_hbm.at[pl.ds(base, _CH)])
            return carry

        lax.fori_loop(0, jw, chunk, 0)

    return posdiff


def _make_msg_scatter(n_pad, e_pad):
    half = n_pad // 2
    jt = e_pad // (_NS * _CH)
    rpt = half // _NS
    arows = half + 128

    @functools.partial(
        pl.kernel,
        out_type=jax.ShapeDtypeStruct((_NC, half, HIDDEN), jnp.float32),
        mesh=_MESH,
        scratch_types=[
            pltpu.VMEM((jt, _CH), jnp.int32),
            pltpu.VMEM((jt, _CH), jnp.int32),
            pltpu.VMEM((1, _CH), jnp.int32),
            pltpu.VMEM((_CH, HIDDEN), jnp.float32),
            pltpu.VMEM((_CH, HIDDEN), jnp.float32),
            pltpu.VMEM_SHARED((arows, HIDDEN), jnp.float32),
            pltpu.SemaphoreType.DMA,
        ],
    )
    def msg_scatter(xl_hbm, w_hbm, row_hbm, col_hbm, zeros_hbm, agg_hbm,
                    ridx, cidx, cloc, xg, wv, aggs, sem1):
        c = lax.axis_index("c")
        s = lax.axis_index("s")
        pltpu.sync_copy(zeros_hbm, aggs.at[pl.ds(s * rpt, rpt)])
        plsc.subcore_barrier()
        pltpu.sync_copy(row_hbm.at[s], ridx)
        pltpu.sync_copy(col_hbm.at[s], cidx)
        lo = c * half
        trash = half + s * 8

        def chunk(j, carry):
            cp = pltpu.async_copy(xl_hbm.at[ridx.at[j]], xg, sem1)
            base = (s * jt + j) * _CH
            pltpu.sync_copy(w_hbm.at[pl.ds(base, _CH)], wv)

            def remap(k, carry2):
                v = cidx[j, pl.ds(k * 16, 16)] - lo
                ok = (v >= 0) & (v < half)
                cloc[0, pl.ds(k * 16, 16)] = jnp.where(ok, v, trash)
                return carry2

            lax.fori_loop(0, _CH // 16, remap, 0, unroll=True)
            cp.wait()

            def row_body(i, carry2):
                for k in range(HIDDEN // 16):
                    sl = pl.ds(k * 16, 16)
                    xg[i, sl] = xg[i, sl] * wv[i, sl]
                return carry2

            lax.fori_loop(0, _CH, row_body, 0)
            pltpu.sync_copy(xg, aggs.at[cloc.at[0]], add=True)
            return carry

        lax.fori_loop(0, jt, chunk, 0)
        plsc.subcore_barrier()
        pltpu.sync_copy(aggs.at[pl.ds(s * rpt, rpt)],
                        agg_hbm.at[c, pl.ds(s * rpt, rpt)])

    return msg_scatter


def _init_body(atom_ref, emb_ref, cf1_ref, h_ref, xl_ref):
    a = atom_ref[...].astype(jnp.float32)
    e0 = emb_ref[0:1, :]
    e1 = emb_ref[1:2, :]
    h = e0 + a * (e1 - e0)
    h_ref[...] = h
    xl_ref[...] = jnp.dot(h, cf1_ref[...], preferred_element_type=jnp.float32)


def _make_filter_body(e_real, be):
    def _filter_body(d3_ref, w1_ref, b1_ref, w2_ref, b2_ref, w_ref):
        i = pl.program_id(0)
        d = d3_ref[...]
        s = jnp.sum(d * d, axis=1, keepdims=True) + 1e-12
        dist = jnp.sqrt(s)
        cc = 0.5 * (jnp.cos(dist * (jnp.pi / CUTOFF)) + 1.0)
        erow = i * be + lax.broadcasted_iota(jnp.int32, (be, 1), 0)
        cc = jnp.where(erow < e_real, cc, 0.0)
        off = lax.broadcasted_iota(jnp.int32, (be, NGAUSS), 1).astype(
            jnp.float32) * _STEP
        attr = jnp.exp(_COEFF * (dist - off) ** 2)
        t = _ssp(jnp.dot(attr, w1_ref[...], preferred_element_type=jnp.float32)
                 + b1_ref[...])
        wmat = (jnp.dot(t, w2_ref[...], preferred_element_type=jnp.float32)
                + b2_ref[...])
        w_ref[...] = wmat * cc

    return _filter_body


def _node_body(h_ref, agg_ref, cf2w_ref, cf2b_ref, linw_ref, linb_ref,
               cf1n_ref, hout_ref, xlout_ref):
    a = agg_ref[0]
    t = _ssp(jnp.dot(a, cf2w_ref[...], preferred_element_type=jnp.float32)
             + cf2b_ref[...])
    t = jnp.dot(t, linw_ref[...], preferred_element_type=jnp.float32) + linb_ref[...]
    h = h_ref[...] + t
    hout_ref[...] = h
    xlout_ref[...] = jnp.dot(h, cf1n_ref[...], preferred_element_type=jnp.float32)


def _readout_body(h_ref, batch_ref, l1w_ref, l1b_ref, ew_ref, qw_ref,
                  eb_ref, qb_ref, e_ref, q_ref, et_ref, qt_ref):
    i = pl.program_id(0)
    hm = _ssp(jnp.dot(h_ref[...], l1w_ref[...], preferred_element_type=jnp.float32)
              + l1b_ref[...])
    e = jnp.dot(hm, ew_ref[...], preferred_element_type=jnp.float32) + eb_ref[...]
    q = jnp.dot(hm, qw_ref[...], preferred_element_type=jnp.float32) + qb_ref[...]
    e_ref[...] = e
    q_ref[...] = q
    b = batch_ref[...]
    gid = lax.broadcasted_iota(jnp.int32, (b.shape[0], NGRAPHS), 1)
    onehot = (b == gid).astype(jnp.float32)
    ec = jnp.sum(onehot * e, axis=0, keepdims=True)
    qc = jnp.sum(onehot * q, axis=0, keepdims=True)

    @pl.when(i == 0)
    def _():
        et_ref[...] = jnp.zeros_like(et_ref)
        qt_ref[...] = jnp.zeros_like(qt_ref)

    et_ref[...] += ec
    qt_ref[...] += qc


def _full(shape):
    return pl.BlockSpec(shape, lambda i: tuple(0 for _ in shape))


def kernel(node_atom, pos, batch, edge_index, emb, mlp_w1, mlp_b1, mlp_w2,
           mlp_b2, cf_lin1_w, cf_lin2_w, cf_lin2_b, lin_w, lin_b, lin1_w,
           lin1_b, e_w, e_b, q_w, q_b):
    n = pos.shape[0]
    e = edge_index.shape[1]
    n_pad = ((n + 2047) // 2048) * 2048
    e_pad = ((e + _NW * _CH - 1) // (_NW * _CH)) * (_NW * _CH)
    half = n_pad // 2

    row = edge_index[0].astype(jnp.int32)
    col = edge_index[1].astype(jnp.int32)
    rowp = jnp.concatenate([row, jnp.zeros((e_pad - e,), jnp.int32)])
    colp = jnp.concatenate([col, jnp.zeros((e_pad - e,), jnp.int32)])
    row32 = rowp.reshape(_NW, e_pad // (_NW * _CH), _CH)
    col32 = colp.reshape(_NW, e_pad // (_NW * _CH), _CH)
    row16 = rowp.reshape(_NS, e_pad // (_NS * _CH), _CH)
    col16 = colp.reshape(_NS, e_pad // (_NS * _CH), _CH)
    pos_pad = jnp.zeros((n_pad, 16), jnp.float32).at[:n, :3].set(pos)
    atom_p = jnp.zeros((n_pad, 1), jnp.int32).at[:n, 0].set(
        node_atom.astype(jnp.int32))
    batch_p = jnp.full((n_pad, 1), NGRAPHS, jnp.int32).at[:n, 0].set(
        batch.astype(jnp.int32))
    zeros_sc = jnp.zeros((half // _NS, HIDDEN), jnp.float32)

    d3 = _make_posdiff(n_pad, e_pad)(pos_pad, row32, col32)
    msg_scatter = _make_msg_scatter(n_pad, e_pad)

    bn = 1024
    be = 2048
    gn = n_pad // bn
    ge = e_pad // be
    hb = half // bn

    h, xl = pl.pallas_call(
        _init_body,
        grid=(gn,),
        in_specs=[pl.BlockSpec((bn, 1), lambda i: (i, 0)),
                  _full((2, HIDDEN)), _full((HIDDEN, HIDDEN))],
        out_specs=[pl.BlockSpec((bn, HIDDEN), lambda i: (i, 0))] * 2,
        out_shape=[jax.ShapeDtypeStruct((n_pad, HIDDEN), jnp.float32)] * 2,
    )(atom_p, emb, cf_lin1_w[0])

    filter_call = pl.pallas_call(
        _make_filter_body(e, be),
        grid=(ge,),
        in_specs=[pl.BlockSpec((be, 16), lambda i: (i, 0)),
                  _full((NGAUSS, NFILT)), _full((1, NFILT)),
                  _full((NFILT, NFILT)), _full((1, NFILT))],
        out_specs=pl.BlockSpec((be, NFILT), lambda i: (i, 0)),
        out_shape=jax.ShapeDtypeStruct((e_pad, NFILT), jnp.float32),
    )

    node_call = pl.pallas_call(
        _node_body,
        grid=(gn,),
        in_specs=[pl.BlockSpec((bn, HIDDEN), lambda i: (i, 0)),
                  pl.BlockSpec((1, bn, HIDDEN), lambda i: (i // hb, i % hb, 0)),
                  _full((HIDDEN, HIDDEN)), _full((1, HIDDEN)),
                  _full((HIDDEN, HIDDEN)), _full((1, HIDDEN)),
                  _full((HIDDEN, HIDDEN))],
        out_specs=[pl.BlockSpec((bn, HIDDEN), lambda i: (i, 0))] * 2,
        out_shape=[jax.ShapeDtypeStruct((n_pad, HIDDEN), jnp.float32)] * 2,
    )

    for l in range(NLAYERS):
        wmat = filter_call(d3, mlp_w1[l], mlp_b1[l].reshape(1, NFILT),
                           mlp_w2[l], mlp_b2[l].reshape(1, NFILT))
        agg = msg_scatter(xl, wmat, row16, col16, zeros_sc)
        cf1n = cf_lin1_w[(l + 1) % NLAYERS]
        h, xl = node_call(h, agg, cf_lin2_w[l],
                          cf_lin2_b[l].reshape(1, HIDDEN), lin_w[l],
                          lin_b[l].reshape(1, HIDDEN), cf1n)

    hh = HIDDEN // 2
    e2, q2, et, qt = pl.pallas_call(
        _readout_body,
        grid=(gn,),
        in_specs=[pl.BlockSpec((bn, HIDDEN), lambda i: (i, 0)),
                  pl.BlockSpec((bn, 1), lambda i: (i, 0)),
                  _full((HIDDEN, hh)), _full((1, hh)),
                  _full((hh, 1)), _full((hh, 1)),
                  _full((1, 1)), _full((1, 1))],
        out_specs=[pl.BlockSpec((bn, 1), lambda i: (i, 0)),
                   pl.BlockSpec((bn, 1), lambda i: (i, 0)),
                   _full((1, NGRAPHS)), _full((1, NGRAPHS))],
        out_shape=[jax.ShapeDtypeStruct((n_pad, 1), jnp.float32),
                   jax.ShapeDtypeStruct((n_pad, 1), jnp.float32),
                   jax.ShapeDtypeStruct((1, NGRAPHS), jnp.float32),
                   jax.ShapeDtypeStruct((1, NGRAPHS), jnp.float32)],
    )(h, batch_p, lin1_w, lin1_b.reshape(1, hh), e_w, q_w,
      e_b.reshape(1, 1), q_b.reshape(1, 1))

    return (e2[:n, 0], q2[:n, 0], et[0], qt[0])
